# baseline (speedup 1.0000x reference)
import jax
import jax.numpy as jnp
from jax.experimental import pallas as pl
from jax.experimental.pallas import tpu as pltpu


_BM = 2048
_SUB = 256
_F8 = jnp.float8_e4m3fn


def _fused_router_kernel(x_ref, w1_ref, b1_ref, w1g_ref, b1g_ref, w2_ref,
                         b2_ref, o_ref, w1t_ref, w1gt_ref, w2t_ref):
    @pl.when(pl.program_id(0) == 0)
    def _prep():
        w1t_ref[...] = w1_ref[...].astype(jnp.bfloat16).T
        w1gt_ref[...] = w1g_ref[...].T.astype(_F8)
        w2t_ref[...] = w2_ref[...].astype(jnp.bfloat16).T

    w1t = w1t_ref[...]
    w1gt = w1gt_ref[...]
    w2t = w2t_ref[...]
    probs = []
    pend = None

    def _tail(gh):
        g, h = gh
        s = jax.nn.sigmoid(g)
        hb = (jnp.maximum(h * s, 0.0)).astype(jnp.bfloat16)
        logits = jnp.dot(hb, w2t,
                         preferred_element_type=jnp.float32) + b2_ref[...]
        m = jnp.max(logits, axis=1, keepdims=True)
        e = jnp.exp(logits - m)
        probs.append(e / jnp.sum(e, axis=1, keepdims=True))

    for k in range(_BM // _SUB):
        x = x_ref[pl.ds(k * _SUB, _SUB), :].astype(jnp.bfloat16)
        x8 = x.astype(_F8)
        g = jnp.dot(x8, w1gt,
                    preferred_element_type=jnp.float32) + b1g_ref[...]
        if pend is not None:
            _tail(pend)
        h = jnp.dot(x, w1t,
                    preferred_element_type=jnp.float32) + b1_ref[...]
        pend = (g, h)
    _tail(pend)
    o_ref[...] = jnp.concatenate(probs, axis=0)


def kernel(input, W1, b1, W1g, b1g, W2, b2):
    tokens, d_in = input.shape
    hidden = W1.shape[0]
    experts = W2.shape[0]
    grid = (tokens // _BM,)
    return pl.pallas_call(
        _fused_router_kernel,
        grid=grid,
        in_specs=[
            pl.BlockSpec((_BM, d_in), lambda i: (i, 0)),
            pl.BlockSpec((hidden, d_in), lambda i: (0, 0)),
            pl.BlockSpec((1, hidden), lambda i: (0, 0)),
            pl.BlockSpec((hidden, d_in), lambda i: (0, 0)),
            pl.BlockSpec((1, hidden), lambda i: (0, 0)),
            pl.BlockSpec((experts, hidden), lambda i: (0, 0)),
            pl.BlockSpec((1, experts), lambda i: (0, 0)),
        ],
        out_specs=pl.BlockSpec((_BM, experts), lambda i: (i, 0)),
        out_shape=jax.ShapeDtypeStruct((tokens, experts), jnp.float32),
        scratch_shapes=[
            pltpu.VMEM((d_in, hidden), jnp.bfloat16),
            pltpu.VMEM((d_in, hidden), _F8),
            pltpu.VMEM((hidden, experts), jnp.bfloat16),
        ],
    )(input, W1, b1.reshape(1, hidden), W1g, b1g.reshape(1, hidden),
      W2, b2.reshape(1, experts))


# speedup vs baseline: 1.0054x; 1.0054x over previous
import jax
import jax.numpy as jnp
from jax.experimental import pallas as pl
from jax.experimental.pallas import tpu as pltpu


_BM = 2048
_SUB = 256
_F8 = jnp.float8_e4m3fn


def _fused_router_kernel(x_ref, w1_ref, b1_ref, w1g_ref, b1g_ref, w2_ref,
                         b2_ref, o_ref, w1t_ref, w1gt_ref, w2t_ref):
    @pl.when(pl.program_id(0) == 0)
    def _prep():
        w1t_ref[...] = w1_ref[...].astype(jnp.bfloat16).T
        w1gt_ref[...] = w1g_ref[...].T.astype(_F8)
        w2t_ref[...] = w2_ref[...].astype(jnp.bfloat16).T

    w1t = w1t_ref[...]
    w1gt = w1gt_ref[...]
    w2t = w2t_ref[...]
    probs = []
    for k in range(_BM // _SUB):
        x = x_ref[pl.ds(k * _SUB, _SUB), :].astype(jnp.bfloat16)
        x8 = x.astype(_F8)
        g = jnp.dot(x8, w1gt,
                    preferred_element_type=jnp.float32) + b1g_ref[...]
        s = jax.nn.sigmoid(g)
        h = jnp.dot(x, w1t,
                    preferred_element_type=jnp.float32) + b1_ref[...]
        hb = (jnp.maximum(h * s, 0.0)).astype(jnp.bfloat16)
        logits = jnp.dot(hb, w2t,
                         preferred_element_type=jnp.float32) + b2_ref[...]
        m = jnp.max(logits, axis=1, keepdims=True)
        e = jnp.exp(logits - m)
        probs.append(e / jnp.sum(e, axis=1, keepdims=True))
    o_ref[...] = jnp.concatenate(probs, axis=0)


def kernel(input, W1, b1, W1g, b1g, W2, b2):
    tokens, d_in = input.shape
    hidden = W1.shape[0]
    experts = W2.shape[0]
    grid = (tokens // _BM,)
    return pl.pallas_call(
        _fused_router_kernel,
        grid=grid,
        in_specs=[
            pl.BlockSpec((_BM, d_in), lambda i: (i, 0)),
            pl.BlockSpec((hidden, d_in), lambda i: (0, 0)),
            pl.BlockSpec((1, hidden), lambda i: (0, 0)),
            pl.BlockSpec((hidden, d_in), lambda i: (0, 0)),
            pl.BlockSpec((1, hidden), lambda i: (0, 0)),
            pl.BlockSpec((experts, hidden), lambda i: (0, 0)),
            pl.BlockSpec((1, experts), lambda i: (0, 0)),
        ],
        out_specs=pl.BlockSpec((_BM, experts), lambda i: (i, 0)),
        out_shape=jax.ShapeDtypeStruct((tokens, experts), jnp.float32),
        scratch_shapes=[
            pltpu.VMEM((d_in, hidden), jnp.bfloat16),
            pltpu.VMEM((d_in, hidden), _F8),
            pltpu.VMEM((hidden, experts), jnp.bfloat16),
        ],
    )(input, W1, b1.reshape(1, hidden), W1g, b1g.reshape(1, hidden),
      W2, b2.reshape(1, experts))
